# Initial kernel scaffold; baseline (speedup 1.0000x reference)
import jax
import jax.numpy as jnp
from jax.experimental import pallas as pl


def kernel(entity_emb, edge_index, edge_type, edge_emb, mess_dropout, q_w, k_w):
    raise NotImplementedError("write your pallas kernel here")


# speedup vs baseline: 2.4092x; 2.4092x over previous
import functools

import jax
import jax.numpy as jnp
from jax import lax
from jax.experimental import pallas as pl
from jax.experimental.pallas import tpu as pltpu
from jax.experimental.pallas import tpu_sc as plsc

N_ENT = 10000
EMB = 128
N_EDGES = 320000
N_REL = 64
N_HOPS = 3

NC = 2
NS = 16
NW = NC * NS
EPT = 10112
CH = 64
CHUNKS = EPT // CH
E_PAD = NW * EPT
CHG = 64
GCHUNKS = EPT // CHG
N_ENT_PAD = 10240
S_ROWS = 10112
SPT = S_ROWS // NS
D_ROWS = 2560
DPT = D_ROWS // NS
EPTILE = N_ENT_PAD // NS
BLK = 1024
NB = E_PAD // BLK
RB = 1024
F32 = jnp.float32


def _sc_gather_body(agg_hbm, q_hbm, tail_hbm, head_hbm, typ_hbm, eemb_hbm,
                    v_out, qh_out, tidx, hidx, yidx, eemb_l, vt, qh, s1, s2):
    cid = lax.axis_index("c")
    sid = lax.axis_index("s")
    wid = cid * NS + sid
    pltpu.sync_copy(eemb_hbm, eemb_l)

    def chunk(c, carry):
        base = wid * EPT + c * CHG
        pltpu.sync_copy(tail_hbm.at[pl.ds(base, CHG)], tidx.at[0])
        pltpu.sync_copy(head_hbm.at[pl.ds(base, CHG)], hidx.at[0])
        pltpu.sync_copy(typ_hbm.at[pl.ds(base, CHG)],
                        yidx.at[0].at[pl.ds(0, CHG)])
        d1 = pltpu.async_copy(agg_hbm.at[tidx.at[0]], vt, s1)
        d2 = pltpu.async_copy(q_hbm.at[hidx.at[0]], qh, s2)
        d1.wait()
        d2.wait()

        def row(r, rc):
            t = yidx[0, pl.ds(r, 16)][0]
            for j in range(EMB // 16):
                sl = pl.ds(16 * j, 16)
                vt[r, sl] = vt[r, sl] * eemb_l[t, sl]
            return rc

        lax.fori_loop(0, CHG, row, 0)
        pltpu.sync_copy(vt, v_out.at[pl.ds(base, CHG)])
        pltpu.sync_copy(qh, qh_out.at[pl.ds(base, CHG)])
        return carry

    lax.fori_loop(0, GCHUNKS, chunk, 0)


_sc_gather = functools.partial(
    pl.kernel,
    out_type=(jax.ShapeDtypeStruct((E_PAD, EMB), F32),
              jax.ShapeDtypeStruct((E_PAD, EMB), F32)),
    mesh=plsc.VectorSubcoreMesh(core_axis_name="c", subcore_axis_name="s"),
    scratch_types=[
        pltpu.VMEM((1, CHG), jnp.int32),
        pltpu.VMEM((1, CHG), jnp.int32),
        pltpu.VMEM((1, CHG + 16), jnp.int32),
        pltpu.VMEM((N_REL, EMB), F32),
        pltpu.VMEM((CHG, EMB), F32),
        pltpu.VMEM((CHG, EMB), F32),
        pltpu.SemaphoreType.DMA,
        pltpu.SemaphoreType.DMA,
    ],
)(_sc_gather_body)


def _sc_scatter_body(att_hbm, gmax_hbm, head_hbm, v_hbm, s2_out, d2_out,
                     attb, hidx, hsc, hshb, gbuf, wbuf, valbuf, vbuf, rows2,
                     obuf, s_sh):
    cid = lax.axis_index("c")
    sid = lax.axis_index("s")
    wid = cid * NS + sid
    zero16 = jnp.zeros((16,), F32)
    lanes = lax.iota(jnp.int32, 16)

    def zrow(r, carry):
        for j in range(EMB // 16):
            obuf[r, pl.ds(16 * j, 16)] = zero16
        return carry

    lax.fori_loop(0, 32, zrow, 0)
    for k in range(SPT // 32):
        pltpu.sync_copy(obuf, s_sh.at[pl.ds(sid * SPT + k * 32, 32)])
    pltpu.sync_copy(obuf.at[pl.ds(0, SPT % 32)],
                    s_sh.at[pl.ds(sid * SPT + (SPT // 32) * 32, SPT % 32)])
    for k in range(DPT // 32):
        pltpu.sync_copy(obuf,
                        s_sh.at[pl.ds(S_ROWS + sid * DPT + k * 32, 32)])
    plsc.subcore_barrier()

    pltpu.sync_copy(gmax_hbm.at[pl.ds(0, 16)], gbuf.at[0])

    def chunk(c, carry):
        base = wid * EPT + c * CH
        pltpu.sync_copy(att_hbm.at[pl.ds(base, CH)], attb.at[0])
        pltpu.sync_copy(head_hbm.at[pl.ds(base, CH)],
                        hidx.at[0].at[pl.ds(0, CH)])
        pltpu.sync_copy(head_hbm.at[pl.ds(base, CH)], hsc.at[0])
        pltpu.sync_copy(v_hbm.at[pl.ds(base, CH)], vbuf)
        g = gbuf[0]
        for gi in range(CH // 16):
            sl = pl.ds(16 * gi, 16)
            a = attb[0, sl]
            eid = base + gi * 16 + lanes
            valid = jnp.where(eid < N_EDGES, 1.0, 0.0).astype(F32)
            wbuf[0, sl] = jnp.exp(a - g) * valid
            valbuf[0, sl] = valid
            h16 = hidx[0, sl]
            hshb[0, sl] = S_ROWS + lax.shift_right_logical(h16, 2)

        def row(r, rc):
            wj = wbuf[0, pl.ds(r, 16)][0]
            vj = valbuf[0, pl.ds(r, 16)][0]
            hj = hidx[0, pl.ds(r, 16)][0]
            b = lax.rem(hj, 4)
            wv = jnp.full((16,), wj, F32)
            for q in range(EMB // 16):
                sl = pl.ds(16 * q, 16)
                vbuf[r, sl] = vbuf[r, sl] * wv
                ff = (b == (q // 2)).astype(F32)
                sj = (wj if q % 2 == 0 else vj) * ff
                rows2[r, sl] = jnp.full((16,), sj, F32)
            return rc

        lax.fori_loop(0, CH, row, 0)
        pltpu.sync_copy(vbuf, s_sh.at[hsc.at[0]], add=True)
        pltpu.sync_copy(rows2, s_sh.at[hshb.at[0]], add=True)
        return carry

    lax.fori_loop(0, CHUNKS, chunk, 0)
    plsc.subcore_barrier()

    def unp_round(rnd, carry):
        pltpu.sync_copy(s_sh.at[pl.ds(S_ROWS + sid * DPT + rnd * 8, 8)],
                        vbuf.at[pl.ds(0, 8)])
        for dr in range(8):
            for slot in range(4):
                den_s = vbuf[dr, pl.ds(32 * slot, 16)][0]
                cnt_s = vbuf[dr, pl.ds(32 * slot + 16, 16)][0]
                tv = jnp.where(
                    lanes == 0, jnp.full((16,), den_s, F32),
                    jnp.where(lanes == 1, jnp.full((16,), cnt_s, F32),
                              zero16))
                obuf[dr * 4 + slot, pl.ds(0, 16)] = tv
        pltpu.sync_copy(obuf,
                        d2_out.at[cid].at[pl.ds(sid * EPTILE + rnd * 32, 32)])
        return carry

    lax.fori_loop(0, DPT // 8, unp_round, 0)

    sl = pl.ds(sid * SPT, SPT)
    pltpu.sync_copy(s_sh.at[sl], s2_out.at[cid].at[sl])


_sc_scatter = functools.partial(
    pl.kernel,
    out_type=(jax.ShapeDtypeStruct((NC, N_ENT_PAD, EMB), F32),
              jax.ShapeDtypeStruct((NC, N_ENT_PAD, EMB), F32)),
    mesh=plsc.VectorSubcoreMesh(core_axis_name="c", subcore_axis_name="s"),
    scratch_types=[
        pltpu.VMEM((1, CH), F32),
        pltpu.VMEM((1, CH + 16), jnp.int32),
        pltpu.VMEM((1, CH), jnp.int32),
        pltpu.VMEM((1, CH), jnp.int32),
        pltpu.VMEM((1, 16), F32),
        pltpu.VMEM((1, CH + 16), F32),
        pltpu.VMEM((1, CH + 16), F32),
        pltpu.VMEM((CH, EMB), F32),
        pltpu.VMEM((CH, EMB), F32),
        pltpu.VMEM((32, EMB), F32),
        pltpu.VMEM_SHARED((S_ROWS + D_ROWS, EMB), F32),
    ],
)(_sc_scatter_body)


def _tc_att_body(v_ref, qh_ref, kw_ref, att_ref, gm_ref):
    right = jnp.tanh(jnp.dot(v_ref[...], kw_ref[...],
                             preferred_element_type=F32))
    s = jnp.sum(qh_ref[...] * right, axis=1)
    att_ref[0] = s.reshape(8, 128)
    m = jnp.max(s)

    @pl.when(pl.program_id(0) == 0)
    def _():
        gm_ref[...] = jnp.full((8, 128), -3e38, F32)

    gm_ref[...] = jnp.maximum(gm_ref[...], m)


def _tc_att(v, qh, k_w):
    return pl.pallas_call(
        _tc_att_body,
        grid=(NB,),
        in_specs=[
            pl.BlockSpec((BLK, EMB), lambda i: (i, 0)),
            pl.BlockSpec((BLK, EMB), lambda i: (i, 0)),
            pl.BlockSpec((EMB, EMB), lambda i: (0, 0)),
        ],
        out_specs=[
            pl.BlockSpec((1, 8, 128), lambda i: (i, 0, 0)),
            pl.BlockSpec((8, 128), lambda i: (0, 0)),
        ],
        out_shape=[
            jax.ShapeDtypeStruct((NB, 8, 128), F32),
            jax.ShapeDtypeStruct((8, 128), F32),
        ],
    )(v, qh, k_w)


def _tc_combine_body(s2_ref, d2_ref, ee_ref, prev_ref, qw_ref,
                     out_ref, agg_ref, q_ref):
    sv = s2_ref[0] + s2_ref[1]
    dd = d2_ref[0] + d2_ref[1]
    den = dd[:, 0:1] + 1e-16
    cnt = jnp.maximum(dd[:, 1:2], 1.0)
    aggv = sv / den / cnt
    n2 = jnp.sum(aggv * aggv, axis=1, keepdims=True)
    aggn = aggv / jnp.maximum(jnp.sqrt(n2), 1e-12)
    out_ref[...] = prev_ref[...] + aggn + ee_ref[...]
    agg_ref[...] = aggn
    q_ref[...] = jnp.dot(aggn, qw_ref[...], preferred_element_type=F32)


def _tc_combine(s2, d2, ee_pad, prev, q_w):
    return pl.pallas_call(
        _tc_combine_body,
        grid=(N_ENT_PAD // RB,),
        in_specs=[
            pl.BlockSpec((NC, RB, EMB), lambda i: (0, i, 0)),
            pl.BlockSpec((NC, RB, EMB), lambda i: (0, i, 0)),
            pl.BlockSpec((RB, EMB), lambda i: (i, 0)),
            pl.BlockSpec((RB, EMB), lambda i: (i, 0)),
            pl.BlockSpec((EMB, EMB), lambda i: (0, 0)),
        ],
        out_specs=[
            pl.BlockSpec((RB, EMB), lambda i: (i, 0)),
            pl.BlockSpec((RB, EMB), lambda i: (i, 0)),
            pl.BlockSpec((RB, EMB), lambda i: (i, 0)),
        ],
        out_shape=[
            jax.ShapeDtypeStruct((N_ENT_PAD, EMB), F32),
            jax.ShapeDtypeStruct((N_ENT_PAD, EMB), F32),
            jax.ShapeDtypeStruct((N_ENT_PAD, EMB), F32),
        ],
    )(s2, d2, ee_pad, prev, q_w)


def _tc_q_body(x_ref, qw_ref, q_ref):
    q_ref[...] = jnp.dot(x_ref[...], qw_ref[...], preferred_element_type=F32)


def _tc_q(x, q_w):
    return pl.pallas_call(
        _tc_q_body,
        grid=(N_ENT_PAD // RB,),
        in_specs=[
            pl.BlockSpec((RB, EMB), lambda i: (i, 0)),
            pl.BlockSpec((EMB, EMB), lambda i: (0, 0)),
        ],
        out_specs=pl.BlockSpec((RB, EMB), lambda i: (i, 0)),
        out_shape=jax.ShapeDtypeStruct((N_ENT_PAD, EMB), F32),
    )(x, q_w)


def kernel(entity_emb, edge_index, edge_type, edge_emb, mess_dropout,
           q_w, k_w):
    ee = entity_emb.astype(F32)
    head = edge_index[0].astype(jnp.int32)
    tail = edge_index[1].astype(jnp.int32)
    typ = edge_type.astype(jnp.int32)
    pad = E_PAD - N_EDGES
    head_p = jnp.concatenate([head, jnp.zeros((pad,), jnp.int32)])
    tail_p = jnp.concatenate([tail, jnp.zeros((pad,), jnp.int32)])
    typ_p = jnp.concatenate([typ, jnp.zeros((pad,), jnp.int32)])

    ee_pad = jnp.concatenate(
        [ee, jnp.zeros((N_ENT_PAD - N_ENT, EMB), F32)], axis=0)
    out = jnp.zeros((N_ENT_PAD, EMB), F32)
    agg = ee_pad
    q = _tc_q(ee_pad, q_w)
    for _ in range(N_HOPS):
        v, qh = _sc_gather(agg, q, tail_p, head_p, typ_p, edge_emb)
        att3, gm = _tc_att(v, qh, k_w)
        s2, d2 = _sc_scatter(att3.reshape(E_PAD), gm.reshape(BLK), head_p, v)
        out, agg, q = _tc_combine(s2, d2, ee_pad, out, q_w)
    return out[:N_ENT]


# speedup vs baseline: 3.5924x; 1.4911x over previous
import functools

import jax
import jax.numpy as jnp
from jax import lax
from jax.experimental import pallas as pl
from jax.experimental.pallas import tpu as pltpu
from jax.experimental.pallas import tpu_sc as plsc

N_ENT = 10000
EMB = 128
N_EDGES = 320000
N_REL = 64
N_HOPS = 3

NC = 2
NS = 16
NW = NC * NS
EPT = 10112
CH = 32
CHUNKS = EPT // CH
E_PAD = NW * EPT
CHG = 64
GCHUNKS = EPT // CHG
N_ENT_PAD = 10240
S_ROWS = 10112
SPT = S_ROWS // NS
D_ROWS = 2560
DPT = D_ROWS // NS
EPTILE = N_ENT_PAD // NS
BLK = 1024
NB = E_PAD // BLK
RB = 1024
F32 = jnp.float32


def _sc_gather_body(agg_hbm, q_hbm, tail_hbm, head_hbm, typ_hbm, eemb_hbm,
                    v_out, qh_out, tidx, hidx, yidx, eemb_l, vt, qh, pv, pq,
                    g0, g1, w0, w1):
    cid = lax.axis_index("c")
    sid = lax.axis_index("s")
    wid = cid * NS + sid
    gsem = (g0, g1)
    wsem = (w0, w1)
    pltpu.sync_copy(eemb_hbm, eemb_l)

    def bsl(s):
        return pl.ds(s * CHG, CHG)

    def fetch(c, s):
        base = wid * EPT + c * CHG
        pltpu.sync_copy(tail_hbm.at[pl.ds(base, CHG)], tidx.at[s])
        pltpu.sync_copy(head_hbm.at[pl.ds(base, CHG)], hidx.at[s])
        pltpu.sync_copy(typ_hbm.at[pl.ds(base, CHG)],
                        yidx.at[s].at[pl.ds(0, CHG)])
        pltpu.async_copy(agg_hbm.at[tidx.at[s]], vt.at[bsl(s)], gsem[s])
        pltpu.async_copy(q_hbm.at[hidx.at[s]], qh.at[bsl(s)], gsem[s])

    def wait_g(s):
        pltpu.make_async_copy(agg_hbm.at[pl.ds(0, CHG)], vt.at[bsl(s)],
                              gsem[s]).wait()
        pltpu.make_async_copy(agg_hbm.at[pl.ds(0, CHG)], qh.at[bsl(s)],
                              gsem[s]).wait()

    def wait_w(s):
        pltpu.make_async_copy(agg_hbm.at[pl.ds(0, CHG)], pv.at[bsl(s)],
                              wsem[s]).wait()
        pltpu.make_async_copy(agg_hbm.at[pl.ds(0, CHG)], pq.at[bsl(s)],
                              wsem[s]).wait()

    def mult(s):
        def row(r, rc):
            t = yidx[s, pl.ds(r, 16)][0]
            rr = s * CHG + r
            for j in range(EMB // 16):
                sl = pl.ds(16 * j, 16)
                pv[rr, sl] = vt[rr, sl] * eemb_l[t, sl]
                pq[rr, sl] = qh[rr, sl]
            return rc

        lax.fori_loop(0, CHG, row, 0)

    def start_w(c, s):
        base = wid * EPT + c * CHG
        pltpu.async_copy(pv.at[bsl(s)], v_out.at[pl.ds(base, CHG)], wsem[s])
        pltpu.async_copy(pq.at[bsl(s)], qh_out.at[pl.ds(base, CHG)], wsem[s])

    fetch(0, 0)
    fetch(1, 1)
    for c in (0, 1):
        wait_g(c)
        mult(c)
        start_w(c, c)
        fetch(c + 2, c)

    def main(c2, carry):
        for s in (0, 1):
            c = 2 * c2 + s
            wait_g(s)
            wait_w(s)
            mult(s)
            start_w(c, s)
            fetch(c + 2, s)
        return carry

    lax.fori_loop(1, GCHUNKS // 2 - 1, main, 0)
    for s in (0, 1):
        c = GCHUNKS - 2 + s
        wait_g(s)
        wait_w(s)
        mult(s)
        start_w(c, s)
    wait_w(0)
    wait_w(1)


_sc_gather = functools.partial(
    pl.kernel,
    out_type=(jax.ShapeDtypeStruct((E_PAD, EMB), F32),
              jax.ShapeDtypeStruct((E_PAD, EMB), F32)),
    mesh=plsc.VectorSubcoreMesh(core_axis_name="c", subcore_axis_name="s"),
    scratch_types=[
        pltpu.VMEM((2, CHG), jnp.int32),
        pltpu.VMEM((2, CHG), jnp.int32),
        pltpu.VMEM((2, CHG + 16), jnp.int32),
        pltpu.VMEM((N_REL, EMB), F32),
        pltpu.VMEM((2 * CHG, EMB), F32),
        pltpu.VMEM((2 * CHG, EMB), F32),
        pltpu.VMEM((2 * CHG, EMB), F32),
        pltpu.VMEM((2 * CHG, EMB), F32),
        pltpu.SemaphoreType.DMA,
        pltpu.SemaphoreType.DMA,
        pltpu.SemaphoreType.DMA,
        pltpu.SemaphoreType.DMA,
    ],
)(_sc_gather_body)


def _sc_scatter_body(att_hbm, gmax_hbm, head_hbm, v_hbm, s2_out, d2_out,
                     attb, hidx, hsc, hshb, gbuf, wbuf, valbuf, vbuf, sbuf,
                     rows2, obuf, s_sh, i0, i1, a0, a1):
    cid = lax.axis_index("c")
    sid = lax.axis_index("s")
    wid = cid * NS + sid
    zero16 = jnp.zeros((16,), F32)
    lanes = lax.iota(jnp.int32, 16)
    isem = (i0, i1)
    asem = (a0, a1)

    def zrow(r, carry):
        for j in range(EMB // 16):
            obuf[r, pl.ds(16 * j, 16)] = zero16
        return carry

    lax.fori_loop(0, 16, zrow, 0)
    for k in range(SPT // 16):
        pltpu.sync_copy(obuf, s_sh.at[pl.ds(sid * SPT + k * 16, 16)])
    pltpu.sync_copy(obuf.at[pl.ds(0, SPT % 16)],
                    s_sh.at[pl.ds(sid * SPT + (SPT // 16) * 16, SPT % 16)])
    for k in range(DPT // 16):
        pltpu.sync_copy(obuf,
                        s_sh.at[pl.ds(S_ROWS + sid * DPT + k * 16, 16)])
    plsc.subcore_barrier()

    pltpu.sync_copy(gmax_hbm.at[pl.ds(0, 16)], gbuf.at[0])

    def bsl(s):
        return pl.ds(s * CH, CH)

    def fetch(c, s):
        base = wid * EPT + c * CH
        s4 = lax.rem(c, 4)
        pltpu.async_copy(att_hbm.at[pl.ds(base, CH)], attb.at[s], isem[s])
        pltpu.async_copy(head_hbm.at[pl.ds(base, CH)],
                         hidx.at[s].at[pl.ds(0, CH)], isem[s])
        pltpu.async_copy(head_hbm.at[pl.ds(base, CH)], hsc.at[s4], isem[s])
        pltpu.async_copy(v_hbm.at[pl.ds(base, CH)], vbuf.at[bsl(s)], isem[s])

    def wait_in(s):
        pltpu.make_async_copy(att_hbm.at[pl.ds(0, CH)], attb.at[s],
                              isem[s]).wait()
        pltpu.make_async_copy(head_hbm.at[pl.ds(0, CH)],
                              hidx.at[s].at[pl.ds(0, CH)], isem[s]).wait()
        pltpu.make_async_copy(head_hbm.at[pl.ds(0, CH)], hsc.at[0],
                              isem[s]).wait()
        pltpu.make_async_copy(v_hbm.at[pl.ds(0, CH)], vbuf.at[bsl(s)],
                              isem[s]).wait()

    def wait_adds(s):
        pltpu.make_async_copy(v_hbm.at[pl.ds(0, CH)], sbuf.at[bsl(s)],
                              asem[s]).wait()
        pltpu.make_async_copy(v_hbm.at[pl.ds(0, CH)], rows2.at[bsl(s)],
                              asem[s]).wait()

    def compute(c, s):
        base = wid * EPT + c * CH
        s4 = lax.rem(c, 4)
        g = gbuf[0]
        for gi in range(CH // 16):
            sl = pl.ds(16 * gi, 16)
            a = attb[s, sl]
            eid = base + gi * 16 + lanes
            valid = jnp.where(eid < N_EDGES, 1.0, 0.0).astype(F32)
            wbuf[s, sl] = jnp.exp(a - g) * valid
            valbuf[s, sl] = valid
            h16 = hidx[s, sl]
            hshb[s4, sl] = S_ROWS + lax.shift_right_logical(h16, 2)

        def row(r, rc):
            wj = wbuf[s, pl.ds(r, 16)][0]
            vj = valbuf[s, pl.ds(r, 16)][0]
            hj = hidx[s, pl.ds(r, 16)][0]
            b = lax.rem(hj, 4)
            wv = jnp.full((16,), wj, F32)
            rr = s * CH + r
            for q in range(EMB // 16):
                sl = pl.ds(16 * q, 16)
                sbuf[rr, sl] = vbuf[rr, sl] * wv
                ff = (b == (q // 2)).astype(F32)
                sj = (wj if q % 2 == 0 else vj) * ff
                rows2[rr, sl] = jnp.full((16,), sj, F32)
            return rc

        lax.fori_loop(0, CH, row, 0)

    def start_adds(c, s):
        s4 = lax.rem(c, 4)
        pltpu.async_copy(sbuf.at[bsl(s)], s_sh.at[hsc.at[s4]], asem[s],
                         add=True)
        pltpu.async_copy(rows2.at[bsl(s)], s_sh.at[hshb.at[s4]], asem[s],
                         add=True)

    fetch(0, 0)
    fetch(1, 1)
    for c in (0, 1):
        wait_in(c)
        compute(c, c)
        start_adds(c, c)
        fetch(c + 2, c)

    def main(c2, carry):
        for s in (0, 1):
            c = 2 * c2 + s
            wait_in(s)
            wait_adds(s)
            compute(c, s)
            start_adds(c, s)
            fetch(c + 2, s)
        return carry

    lax.fori_loop(1, CHUNKS // 2 - 1, main, 0)
    for s in (0, 1):
        c = CHUNKS - 2 + s
        wait_in(s)
        wait_adds(s)
        compute(c, s)
        start_adds(c, s)
    wait_adds(0)
    wait_adds(1)
    plsc.subcore_barrier()

    def unp_round(rnd, carry):
        pltpu.sync_copy(s_sh.at[pl.ds(S_ROWS + sid * DPT + rnd * 8, 8)],
                        vbuf.at[pl.ds(0, 8)])
        for half in range(2):
            for dr in range(4):
                for slot in range(4):
                    src_r = half * 4 + dr
                    den_s = vbuf[src_r, pl.ds(32 * slot, 16)][0]
                    cnt_s = vbuf[src_r, pl.ds(32 * slot + 16, 16)][0]
                    tv = jnp.where(
                        lanes == 0, jnp.full((16,), den_s, F32),
                        jnp.where(lanes == 1, jnp.full((16,), cnt_s, F32),
                                  zero16))
                    obuf[dr * 4 + slot, pl.ds(0, 16)] = tv
            pltpu.sync_copy(
                obuf,
                d2_out.at[cid].at[pl.ds(sid * EPTILE + rnd * 32 + half * 16,
                                        16)])
        return carry

    lax.fori_loop(0, DPT // 8, unp_round, 0)

    sl = pl.ds(sid * SPT, SPT)
    pltpu.sync_copy(s_sh.at[sl], s2_out.at[cid].at[sl])


_sc_scatter = functools.partial(
    pl.kernel,
    out_type=(jax.ShapeDtypeStruct((NC, N_ENT_PAD, EMB), F32),
              jax.ShapeDtypeStruct((NC, N_ENT_PAD, EMB), F32)),
    mesh=plsc.VectorSubcoreMesh(core_axis_name="c", subcore_axis_name="s"),
    scratch_types=[
        pltpu.VMEM((2, CH), F32),
        pltpu.VMEM((2, CH + 16), jnp.int32),
        pltpu.VMEM((4, CH), jnp.int32),
        pltpu.VMEM((4, CH), jnp.int32),
        pltpu.VMEM((1, 16), F32),
        pltpu.VMEM((2, CH + 16), F32),
        pltpu.VMEM((2, CH + 16), F32),
        pltpu.VMEM((2 * CH, EMB), F32),
        pltpu.VMEM((2 * CH, EMB), F32),
        pltpu.VMEM((2 * CH, EMB), F32),
        pltpu.VMEM((16, EMB), F32),
        pltpu.VMEM_SHARED((S_ROWS + D_ROWS, EMB), F32),
        pltpu.SemaphoreType.DMA,
        pltpu.SemaphoreType.DMA,
        pltpu.SemaphoreType.DMA,
        pltpu.SemaphoreType.DMA,
    ],
)(_sc_scatter_body)


def _tc_att_body(v_ref, qh_ref, kw_ref, att_ref, gm_ref):
    right = jnp.tanh(jnp.dot(v_ref[...], kw_ref[...],
                             preferred_element_type=F32))
    s = jnp.sum(qh_ref[...] * right, axis=1)
    att_ref[0] = s.reshape(8, 128)
    m = jnp.max(s)

    @pl.when(pl.program_id(0) == 0)
    def _():
        gm_ref[...] = jnp.full((8, 128), -3e38, F32)

    gm_ref[...] = jnp.maximum(gm_ref[...], m)


def _tc_att(v, qh, k_w):
    return pl.pallas_call(
        _tc_att_body,
        grid=(NB,),
        in_specs=[
            pl.BlockSpec((BLK, EMB), lambda i: (i, 0)),
            pl.BlockSpec((BLK, EMB), lambda i: (i, 0)),
            pl.BlockSpec((EMB, EMB), lambda i: (0, 0)),
        ],
        out_specs=[
            pl.BlockSpec((1, 8, 128), lambda i: (i, 0, 0)),
            pl.BlockSpec((8, 128), lambda i: (0, 0)),
        ],
        out_shape=[
            jax.ShapeDtypeStruct((NB, 8, 128), F32),
            jax.ShapeDtypeStruct((8, 128), F32),
        ],
    )(v, qh, k_w)


def _tc_combine_body(s2_ref, d2_ref, ee_ref, prev_ref, qw_ref,
                     out_ref, agg_ref, q_ref):
    sv = s2_ref[0] + s2_ref[1]
    dd = d2_ref[0] + d2_ref[1]
    den = dd[:, 0:1] + 1e-16
    cnt = jnp.maximum(dd[:, 1:2], 1.0)
    aggv = sv / den / cnt
    n2 = jnp.sum(aggv * aggv, axis=1, keepdims=True)
    aggn = aggv / jnp.maximum(jnp.sqrt(n2), 1e-12)
    out_ref[...] = prev_ref[...] + aggn + ee_ref[...]
    agg_ref[...] = aggn
    q_ref[...] = jnp.dot(aggn, qw_ref[...], preferred_element_type=F32)


def _tc_combine(s2, d2, ee_pad, prev, q_w):
    return pl.pallas_call(
        _tc_combine_body,
        grid=(N_ENT_PAD // RB,),
        in_specs=[
            pl.BlockSpec((NC, RB, EMB), lambda i: (0, i, 0)),
            pl.BlockSpec((NC, RB, EMB), lambda i: (0, i, 0)),
            pl.BlockSpec((RB, EMB), lambda i: (i, 0)),
            pl.BlockSpec((RB, EMB), lambda i: (i, 0)),
            pl.BlockSpec((EMB, EMB), lambda i: (0, 0)),
        ],
        out_specs=[
            pl.BlockSpec((RB, EMB), lambda i: (i, 0)),
            pl.BlockSpec((RB, EMB), lambda i: (i, 0)),
            pl.BlockSpec((RB, EMB), lambda i: (i, 0)),
        ],
        out_shape=[
            jax.ShapeDtypeStruct((N_ENT_PAD, EMB), F32),
            jax.ShapeDtypeStruct((N_ENT_PAD, EMB), F32),
            jax.ShapeDtypeStruct((N_ENT_PAD, EMB), F32),
        ],
    )(s2, d2, ee_pad, prev, q_w)


def _tc_q_body(x_ref, qw_ref, q_ref):
    q_ref[...] = jnp.dot(x_ref[...], qw_ref[...], preferred_element_type=F32)


def _tc_q(x, q_w):
    return pl.pallas_call(
        _tc_q_body,
        grid=(N_ENT_PAD // RB,),
        in_specs=[
            pl.BlockSpec((RB, EMB), lambda i: (i, 0)),
            pl.BlockSpec((EMB, EMB), lambda i: (0, 0)),
        ],
        out_specs=pl.BlockSpec((RB, EMB), lambda i: (i, 0)),
        out_shape=jax.ShapeDtypeStruct((N_ENT_PAD, EMB), F32),
    )(x, q_w)


def kernel(entity_emb, edge_index, edge_type, edge_emb, mess_dropout,
           q_w, k_w):
    ee = entity_emb.astype(F32)
    head = edge_index[0].astype(jnp.int32)
    tail = edge_index[1].astype(jnp.int32)
    typ = edge_type.astype(jnp.int32)
    pad = E_PAD - N_EDGES
    head_p = jnp.concatenate([head, jnp.zeros((pad,), jnp.int32)])
    tail_p = jnp.concatenate([tail, jnp.zeros((pad,), jnp.int32)])
    typ_p = jnp.concatenate([typ, jnp.zeros((pad,), jnp.int32)])

    ee_pad = jnp.concatenate(
        [ee, jnp.zeros((N_ENT_PAD - N_ENT, EMB), F32)], axis=0)
    out = jnp.zeros((N_ENT_PAD, EMB), F32)
    agg = ee_pad
    q = _tc_q(ee_pad, q_w)
    for _ in range(N_HOPS):
        v, qh = _sc_gather(agg, q, tail_p, head_p, typ_p, edge_emb)
        att3, gm = _tc_att(v, qh, k_w)
        s2, d2 = _sc_scatter(att3.reshape(E_PAD), gm.reshape(BLK), head_p, v)
        out, agg, q = _tc_combine(s2, d2, ee_pad, out, q_w)
    return out[:N_ENT]


# speedup vs baseline: 3.9594x; 1.1022x over previous
import functools

import jax
import jax.numpy as jnp
from jax import lax
from jax.experimental import pallas as pl
from jax.experimental.pallas import tpu as pltpu
from jax.experimental.pallas import tpu_sc as plsc

N_ENT = 10000
EMB = 128
N_EDGES = 320000
N_REL = 64
N_HOPS = 3

NC = 2
NS = 16
NW = NC * NS
EPT = 10112
CH = 32
CHUNKS = EPT // CH
E_PAD = NW * EPT
CHG = 64
GCHUNKS = EPT // CHG
N_ENT_PAD = 10240
S_ROWS = 10112
SPT = S_ROWS // NS
D_ROWS = 2560
DPT = D_ROWS // NS
EPTILE = N_ENT_PAD // NS
BLK = 1024
NB = E_PAD // BLK
RB = 1024
F32 = jnp.float32


def _sc_gather_body(agg_hbm, q_hbm, tail_hbm, head_hbm, typ_hbm, eemb_hbm,
                    v_out, qh_out, tidx, hidx, yidx, eemb_l, vt, qh, pv, pq,
                    g0, g1, w0, w1):
    cid = lax.axis_index("c")
    sid = lax.axis_index("s")
    wid = cid * NS + sid
    gsem = (g0, g1)
    wsem = (w0, w1)
    pltpu.sync_copy(eemb_hbm, eemb_l)
    tbase = wid * EPT
    pltpu.sync_copy(tail_hbm.at[pl.ds(tbase, EPT)], tidx.at[pl.ds(0, EPT)])
    pltpu.sync_copy(head_hbm.at[pl.ds(tbase, EPT)], hidx.at[pl.ds(0, EPT)])
    pltpu.sync_copy(typ_hbm.at[pl.ds(tbase, EPT)], yidx.at[pl.ds(0, EPT)])

    def bsl(s):
        return pl.ds(s * CHG, CHG)

    def fetch(c, s):
        csl = pl.ds(c * CHG, CHG)
        pltpu.async_copy(agg_hbm.at[tidx.at[csl]], vt.at[bsl(s)], gsem[s])
        pltpu.async_copy(q_hbm.at[hidx.at[csl]], qh.at[bsl(s)], gsem[s])

    def wait_g(s):
        pltpu.make_async_copy(agg_hbm.at[pl.ds(0, CHG)], vt.at[bsl(s)],
                              gsem[s]).wait()
        pltpu.make_async_copy(agg_hbm.at[pl.ds(0, CHG)], qh.at[bsl(s)],
                              gsem[s]).wait()

    def wait_w(s):
        pltpu.make_async_copy(agg_hbm.at[pl.ds(0, CHG)], pv.at[bsl(s)],
                              wsem[s]).wait()
        pltpu.make_async_copy(agg_hbm.at[pl.ds(0, CHG)], pq.at[bsl(s)],
                              wsem[s]).wait()

    def mult(c, s):
        cb = c * CHG

        def row(r, rc):
            t = yidx[pl.ds(cb + r, 16)][0]
            rr = s * CHG + r
            for j in range(EMB // 16):
                sl = pl.ds(16 * j, 16)
                pv[rr, sl] = vt[rr, sl] * eemb_l[t, sl]
                pq[rr, sl] = qh[rr, sl]
            return rc

        lax.fori_loop(0, CHG, row, 0)

    def start_w(c, s):
        base = wid * EPT + c * CHG
        pltpu.async_copy(pv.at[bsl(s)], v_out.at[pl.ds(base, CHG)], wsem[s])
        pltpu.async_copy(pq.at[bsl(s)], qh_out.at[pl.ds(base, CHG)], wsem[s])

    fetch(0, 0)
    fetch(1, 1)
    for c in (0, 1):
        wait_g(c)
        mult(c, c)
        start_w(c, c)
        fetch(c + 2, c)

    def main(c2, carry):
        for s in (0, 1):
            c = 2 * c2 + s
            wait_g(s)
            wait_w(s)
            mult(c, s)
            start_w(c, s)
            fetch(c + 2, s)
        return carry

    lax.fori_loop(1, GCHUNKS // 2 - 1, main, 0)
    for s in (0, 1):
        c = GCHUNKS - 2 + s
        wait_g(s)
        wait_w(s)
        mult(c, s)
        start_w(c, s)
    wait_w(0)
    wait_w(1)


_sc_gather = functools.partial(
    pl.kernel,
    out_type=(jax.ShapeDtypeStruct((E_PAD, EMB), F32),
              jax.ShapeDtypeStruct((E_PAD, EMB), F32)),
    mesh=plsc.VectorSubcoreMesh(core_axis_name="c", subcore_axis_name="s"),
    scratch_types=[
        pltpu.VMEM((EPT,), jnp.int32),
        pltpu.VMEM((EPT,), jnp.int32),
        pltpu.VMEM((EPT + 16,), jnp.int32),
        pltpu.VMEM((N_REL, EMB), F32),
        pltpu.VMEM((2 * CHG, EMB), F32),
        pltpu.VMEM((2 * CHG, EMB), F32),
        pltpu.VMEM((2 * CHG, EMB), F32),
        pltpu.VMEM((2 * CHG, EMB), F32),
        pltpu.SemaphoreType.DMA,
        pltpu.SemaphoreType.DMA,
        pltpu.SemaphoreType.DMA,
        pltpu.SemaphoreType.DMA,
    ],
)(_sc_gather_body)


def _sc_scatter_body(att_hbm, gmax_hbm, head_hbm, v_hbm,
                     s2_out, d2_out,
                     attb, hidx, hc_all, hh_all,
                     gbuf, wbuf, valbuf, vbuf, sbuf,
                     rows2, obuf, s_sh, i0, i1, a0, a1):
    cid = lax.axis_index("c")
    sid = lax.axis_index("s")
    wid = cid * NS + sid
    zero16 = jnp.zeros((16,), F32)
    lanes = lax.iota(jnp.int32, 16)
    isem = (i0, i1)
    asem = (a0, a1)

    def zrow(r, carry):
        for j in range(EMB // 16):
            obuf[r, pl.ds(16 * j, 16)] = zero16
        return carry

    lax.fori_loop(0, 16, zrow, 0)
    for k in range(SPT // 16):
        pltpu.sync_copy(obuf, s_sh.at[pl.ds(sid * SPT + k * 16, 16)])
    pltpu.sync_copy(obuf.at[pl.ds(0, SPT % 16)],
                    s_sh.at[pl.ds(sid * SPT + (SPT // 16) * 16, SPT % 16)])
    for k in range(DPT // 16):
        pltpu.sync_copy(obuf,
                        s_sh.at[pl.ds(S_ROWS + sid * DPT + k * 16, 16)])
    plsc.subcore_barrier()

    pltpu.sync_copy(gmax_hbm.at[pl.ds(0, 16)], gbuf.at[0])

    def bsl(s):
        return pl.ds(s * CH, CH)

    def slot(c):
        return lax.rem(c, 4)

    def fetch(c, s):
        base = wid * EPT + c * CH
        pltpu.async_copy(att_hbm.at[pl.ds(base, CH)], attb.at[s], isem[s])
        pltpu.async_copy(head_hbm.at[pl.ds(base, CH)],
                         hidx.at[s].at[pl.ds(0, CH)], isem[s])
        pltpu.async_copy(head_hbm.at[pl.ds(base, CH)], hc_all.at[slot(c)],
                         isem[s])
        pltpu.async_copy(v_hbm.at[pl.ds(base, CH)], vbuf.at[bsl(s)], isem[s])

    def wait_in(s):
        pltpu.make_async_copy(att_hbm.at[pl.ds(0, CH)], attb.at[s],
                              isem[s]).wait()
        pltpu.make_async_copy(head_hbm.at[pl.ds(0, CH)],
                              hidx.at[s].at[pl.ds(0, CH)], isem[s]).wait()
        pltpu.make_async_copy(head_hbm.at[pl.ds(0, CH)], hc_all.at[0],
                              isem[s]).wait()
        pltpu.make_async_copy(v_hbm.at[pl.ds(0, CH)], vbuf.at[bsl(s)],
                              isem[s]).wait()

    def wait_adds(s):
        pltpu.make_async_copy(v_hbm.at[pl.ds(0, CH)], sbuf.at[bsl(s)],
                              asem[s]).wait()
        pltpu.make_async_copy(v_hbm.at[pl.ds(0, CH)], rows2.at[bsl(s)],
                              asem[s]).wait()

    def compute(c, s):
        base = wid * EPT + c * CH
        dr = slot(c)
        g = gbuf[0]
        for gi in range(CH // 16):
            sl = pl.ds(16 * gi, 16)
            a = attb[s, sl]
            eid = base + gi * 16 + lanes
            valid = jnp.where(eid < N_EDGES, 1.0, 0.0).astype(F32)
            wbuf[s, sl] = jnp.exp(a - g) * valid
            valbuf[s, sl] = valid
            h16 = hidx[s, sl]
            hh_all[dr, sl] = S_ROWS + lax.shift_right_logical(h16, 2)

        def row(r, rc):
            wj = wbuf[s, pl.ds(r, 16)][0]
            vj = valbuf[s, pl.ds(r, 16)][0]
            hj = hidx[s, pl.ds(r, 16)][0]
            b = lax.rem(hj, 4)
            wv = jnp.full((16,), wj, F32)
            rr = s * CH + r
            for q in range(EMB // 16):
                sl = pl.ds(16 * q, 16)
                sbuf[rr, sl] = vbuf[rr, sl] * wv
                ff = (b == (q // 2)).astype(F32)
                sj = (wj if q % 2 == 0 else vj) * ff
                rows2[rr, sl] = jnp.full((16,), sj, F32)
            return rc

        lax.fori_loop(0, CH, row, 0)

    def start_adds(c, s):
        dr = slot(c)
        pltpu.async_copy(sbuf.at[bsl(s)], s_sh.at[hc_all.at[dr]], asem[s],
                         add=True)
        pltpu.async_copy(rows2.at[bsl(s)], s_sh.at[hh_all.at[dr]], asem[s],
                         add=True)

    fetch(0, 0)
    fetch(1, 1)
    for c in (0, 1):
        wait_in(c)
        compute(c, c)
        start_adds(c, c)
        fetch(c + 2, c)

    def main(c2, carry):
        for s in (0, 1):
            c = 2 * c2 + s
            wait_in(s)
            wait_adds(s)
            compute(c, s)
            start_adds(c, s)
            fetch(c + 2, s)
        return carry

    lax.fori_loop(1, CHUNKS // 2 - 1, main, 0)
    for s in (0, 1):
        c = CHUNKS - 2 + s
        wait_in(s)
        wait_adds(s)
        compute(c, s)
        start_adds(c, s)
    wait_adds(0)
    wait_adds(1)
    plsc.subcore_barrier()

    def unp_round(rnd, carry):
        pltpu.sync_copy(s_sh.at[pl.ds(S_ROWS + sid * DPT + rnd * 8, 8)],
                        vbuf.at[pl.ds(0, 8)])
        for half in range(2):
            for dr in range(4):
                for slot in range(4):
                    src_r = half * 4 + dr
                    den_s = vbuf[src_r, pl.ds(32 * slot, 16)][0]
                    cnt_s = vbuf[src_r, pl.ds(32 * slot + 16, 16)][0]
                    tv = jnp.where(
                        lanes == 0, jnp.full((16,), den_s, F32),
                        jnp.where(lanes == 1, jnp.full((16,), cnt_s, F32),
                                  zero16))
                    obuf[dr * 4 + slot, pl.ds(0, 16)] = tv
            pltpu.sync_copy(
                obuf,
                d2_out.at[cid].at[pl.ds(sid * EPTILE + rnd * 32 + half * 16,
                                        16)])
        return carry

    lax.fori_loop(0, DPT // 8, unp_round, 0)

    sl = pl.ds(sid * SPT, SPT)
    pltpu.sync_copy(s_sh.at[sl], s2_out.at[cid].at[sl])


_sc_scatter = functools.partial(
    pl.kernel,
    out_type=(jax.ShapeDtypeStruct((NC, N_ENT_PAD, EMB), F32),
              jax.ShapeDtypeStruct((NC, N_ENT_PAD, EMB), F32)),
    mesh=plsc.VectorSubcoreMesh(core_axis_name="c", subcore_axis_name="s"),
    scratch_types=[
        pltpu.VMEM((2, CH), F32),
        pltpu.VMEM((2, CH + 16), jnp.int32),
        pltpu.VMEM((4, CH), jnp.int32),
        pltpu.VMEM((4, CH), jnp.int32),
        pltpu.VMEM((1, 16), F32),
        pltpu.VMEM((2, CH + 16), F32),
        pltpu.VMEM((2, CH + 16), F32),
        pltpu.VMEM((2 * CH, EMB), F32),
        pltpu.VMEM((2 * CH, EMB), F32),
        pltpu.VMEM((2 * CH, EMB), F32),
        pltpu.VMEM((16, EMB), F32),
        pltpu.VMEM_SHARED((S_ROWS + D_ROWS, EMB), F32),
        pltpu.SemaphoreType.DMA,
        pltpu.SemaphoreType.DMA,
        pltpu.SemaphoreType.DMA,
        pltpu.SemaphoreType.DMA,
    ],
)(_sc_scatter_body)


def _tc_att_body(v_ref, qh_ref, kw_ref, att_ref, gm_ref):
    right = jnp.tanh(jnp.dot(v_ref[...], kw_ref[...],
                             preferred_element_type=F32))
    s = jnp.sum(qh_ref[...] * right, axis=1)
    att_ref[0] = s.reshape(8, 128)
    m = jnp.max(s)

    @pl.when(pl.program_id(0) == 0)
    def _():
        gm_ref[...] = jnp.full((8, 128), -3e38, F32)

    gm_ref[...] = jnp.maximum(gm_ref[...], m)


def _tc_att(v, qh, k_w):
    return pl.pallas_call(
        _tc_att_body,
        grid=(NB,),
        in_specs=[
            pl.BlockSpec((BLK, EMB), lambda i: (i, 0)),
            pl.BlockSpec((BLK, EMB), lambda i: (i, 0)),
            pl.BlockSpec((EMB, EMB), lambda i: (0, 0)),
        ],
        out_specs=[
            pl.BlockSpec((1, 8, 128), lambda i: (i, 0, 0)),
            pl.BlockSpec((8, 128), lambda i: (0, 0)),
        ],
        out_shape=[
            jax.ShapeDtypeStruct((NB, 8, 128), F32),
            jax.ShapeDtypeStruct((8, 128), F32),
        ],
    )(v, qh, k_w)


def _tc_combine_body(s2_ref, d2_ref, ee_ref, prev_ref, qw_ref,
                     out_ref, agg_ref, q_ref):
    sv = s2_ref[0] + s2_ref[1]
    dd = d2_ref[0] + d2_ref[1]
    den = dd[:, 0:1] + 1e-16
    cnt = jnp.maximum(dd[:, 1:2], 1.0)
    aggv = sv / den / cnt
    n2 = jnp.sum(aggv * aggv, axis=1, keepdims=True)
    aggn = aggv / jnp.maximum(jnp.sqrt(n2), 1e-12)
    out_ref[...] = prev_ref[...] + aggn + ee_ref[...]
    agg_ref[...] = aggn
    q_ref[...] = jnp.dot(aggn, qw_ref[...], preferred_element_type=F32)


def _tc_combine(s2, d2, ee_pad, prev, q_w):
    return pl.pallas_call(
        _tc_combine_body,
        grid=(N_ENT_PAD // RB,),
        in_specs=[
            pl.BlockSpec((NC, RB, EMB), lambda i: (0, i, 0)),
            pl.BlockSpec((NC, RB, EMB), lambda i: (0, i, 0)),
            pl.BlockSpec((RB, EMB), lambda i: (i, 0)),
            pl.BlockSpec((RB, EMB), lambda i: (i, 0)),
            pl.BlockSpec((EMB, EMB), lambda i: (0, 0)),
        ],
        out_specs=[
            pl.BlockSpec((RB, EMB), lambda i: (i, 0)),
            pl.BlockSpec((RB, EMB), lambda i: (i, 0)),
            pl.BlockSpec((RB, EMB), lambda i: (i, 0)),
        ],
        out_shape=[
            jax.ShapeDtypeStruct((N_ENT_PAD, EMB), F32),
            jax.ShapeDtypeStruct((N_ENT_PAD, EMB), F32),
            jax.ShapeDtypeStruct((N_ENT_PAD, EMB), F32),
        ],
    )(s2, d2, ee_pad, prev, q_w)


def _tc_q_body(x_ref, qw_ref, q_ref):
    q_ref[...] = jnp.dot(x_ref[...], qw_ref[...], preferred_element_type=F32)


def _tc_q(x, q_w):
    return pl.pallas_call(
        _tc_q_body,
        grid=(N_ENT_PAD // RB,),
        in_specs=[
            pl.BlockSpec((RB, EMB), lambda i: (i, 0)),
            pl.BlockSpec((EMB, EMB), lambda i: (0, 0)),
        ],
        out_specs=pl.BlockSpec((RB, EMB), lambda i: (i, 0)),
        out_shape=jax.ShapeDtypeStruct((N_ENT_PAD, EMB), F32),
    )(x, q_w)


def kernel(entity_emb, edge_index, edge_type, edge_emb, mess_dropout,
           q_w, k_w):
    ee = entity_emb.astype(F32)
    head = edge_index[0].astype(jnp.int32)
    tail = edge_index[1].astype(jnp.int32)
    typ = edge_type.astype(jnp.int32)
    pad = E_PAD - N_EDGES
    head_p = jnp.concatenate([head, jnp.zeros((pad,), jnp.int32)])
    tail_p = jnp.concatenate([tail, jnp.zeros((pad,), jnp.int32)])
    typ_p = jnp.concatenate([typ, jnp.zeros((pad,), jnp.int32)])

    ee_pad = jnp.concatenate(
        [ee, jnp.zeros((N_ENT_PAD - N_ENT, EMB), F32)], axis=0)
    out = jnp.zeros((N_ENT_PAD, EMB), F32)
    agg = ee_pad
    q = _tc_q(ee_pad, q_w)
    for _ in range(N_HOPS):
        v, qh = _sc_gather(agg, q, tail_p, head_p, typ_p, edge_emb)
        att3, gm = _tc_att(v, qh, k_w)
        s2, d2 = _sc_scatter(att3.reshape(E_PAD), gm.reshape(BLK), head_p, v)
        out, agg, q = _tc_combine(s2, d2, ee_pad, out, q_w)
    return out[:N_ENT]


# speedup vs baseline: 6.6744x; 1.6857x over previous
import functools

import jax
import jax.numpy as jnp
from jax import lax
from jax.experimental import pallas as pl
from jax.experimental.pallas import tpu as pltpu
from jax.experimental.pallas import tpu_sc as plsc

N_ENT = 10000
EMB = 128
N_EDGES = 320000
N_REL = 64
N_HOPS = 3

NC = 2
NS = 16
NW = NC * NS
EPT = 10112
CH = 32
CHUNKS = EPT // CH
E_PAD = NW * EPT
CHG = 64
GCHUNKS = EPT // CHG
N_ENT_PAD = 10240
S_ROWS = 10112
SPT = S_ROWS // NS
D_ROWS = 2560
DPT = D_ROWS // NS
EPTILE = N_ENT_PAD // NS
BLK = 1024
NB = E_PAD // BLK
RB = 1024
F32 = jnp.float32


def _sc_gather_body(agg_hbm, q_hbm, tail_hbm, head_hbm, typ_hbm, eemb_hbm,
                    v_out, qh_out, tidx, hidx, yidx, eemb_l, vt, qh, pv, pq,
                    g0, g1, w0, w1):
    cid = lax.axis_index("c")
    sid = lax.axis_index("s")
    wid = cid * NS + sid
    gsem = (g0, g1)
    wsem = (w0, w1)
    pltpu.sync_copy(eemb_hbm, eemb_l)
    tbase = wid * EPT
    pltpu.sync_copy(tail_hbm.at[pl.ds(tbase, EPT)], tidx.at[pl.ds(0, EPT)])
    pltpu.sync_copy(head_hbm.at[pl.ds(tbase, EPT)], hidx.at[pl.ds(0, EPT)])
    pltpu.sync_copy(typ_hbm.at[pl.ds(tbase, EPT)], yidx.at[pl.ds(0, EPT)])

    def bsl(s):
        return pl.ds(s * CHG, CHG)

    def fetch(c, s):
        csl = pl.ds(c * CHG, CHG)
        pltpu.async_copy(agg_hbm.at[tidx.at[csl]], vt.at[bsl(s)], gsem[s])
        pltpu.async_copy(q_hbm.at[hidx.at[csl]], qh.at[bsl(s)], gsem[s])

    def wait_g(s):
        pltpu.make_async_copy(agg_hbm.at[pl.ds(0, CHG)], vt.at[bsl(s)],
                              gsem[s]).wait()
        pltpu.make_async_copy(agg_hbm.at[pl.ds(0, CHG)], qh.at[bsl(s)],
                              gsem[s]).wait()

    def wait_w(s):
        pltpu.make_async_copy(agg_hbm.at[pl.ds(0, CHG)], pv.at[bsl(s)],
                              wsem[s]).wait()
        pltpu.make_async_copy(agg_hbm.at[pl.ds(0, CHG)], pq.at[bsl(s)],
                              wsem[s]).wait()

    def mult(c, s):
        cb = c * CHG

        @functools.partial(plsc.parallel_loop, 0, CHG, unroll=4)
        def row(r):
            t = yidx[pl.ds(cb + r, 16)][0]
            rr = s * CHG + r
            for j in range(EMB // 16):
                sl = pl.ds(16 * j, 16)
                pv[rr, sl] = vt[rr, sl] * eemb_l[t, sl]
                pq[rr, sl] = qh[rr, sl]

    def start_w(c, s):
        base = wid * EPT + c * CHG
        pltpu.async_copy(pv.at[bsl(s)], v_out.at[pl.ds(base, CHG)], wsem[s])
        pltpu.async_copy(pq.at[bsl(s)], qh_out.at[pl.ds(base, CHG)], wsem[s])

    fetch(0, 0)
    fetch(1, 1)
    for c in (0, 1):
        wait_g(c)
        mult(c, c)
        start_w(c, c)
        fetch(c + 2, c)

    def main(c2, carry):
        for s in (0, 1):
            c = 2 * c2 + s
            wait_g(s)
            wait_w(s)
            mult(c, s)
            start_w(c, s)
            fetch(c + 2, s)
        return carry

    lax.fori_loop(1, GCHUNKS // 2 - 1, main, 0)
    for s in (0, 1):
        c = GCHUNKS - 2 + s
        wait_g(s)
        wait_w(s)
        mult(c, s)
        start_w(c, s)
    wait_w(0)
    wait_w(1)


_sc_gather = functools.partial(
    pl.kernel,
    out_type=(jax.ShapeDtypeStruct((E_PAD, EMB), F32),
              jax.ShapeDtypeStruct((E_PAD, EMB), F32)),
    mesh=plsc.VectorSubcoreMesh(core_axis_name="c", subcore_axis_name="s"),
    scratch_types=[
        pltpu.VMEM((EPT,), jnp.int32),
        pltpu.VMEM((EPT,), jnp.int32),
        pltpu.VMEM((EPT + 16,), jnp.int32),
        pltpu.VMEM((N_REL, EMB), F32),
        pltpu.VMEM((2 * CHG, EMB), F32),
        pltpu.VMEM((2 * CHG, EMB), F32),
        pltpu.VMEM((2 * CHG, EMB), F32),
        pltpu.VMEM((2 * CHG, EMB), F32),
        pltpu.SemaphoreType.DMA,
        pltpu.SemaphoreType.DMA,
        pltpu.SemaphoreType.DMA,
        pltpu.SemaphoreType.DMA,
    ],
)(_sc_gather_body)


def _sc_scatter_body(att_hbm, gmax_hbm, head_hbm, v_hbm,
                     s2_out, d2_out,
                     attb, hidx, hc_all, hh_all,
                     gbuf, wbuf, valbuf, vbuf, sbuf,
                     rows2, obuf, s_sh, i0, i1, a0, a1):
    cid = lax.axis_index("c")
    sid = lax.axis_index("s")
    wid = cid * NS + sid
    zero16 = jnp.zeros((16,), F32)
    lanes = lax.iota(jnp.int32, 16)
    isem = (i0, i1)
    asem = (a0, a1)

    def zrow(r, carry):
        for j in range(EMB // 16):
            obuf[r, pl.ds(16 * j, 16)] = zero16
        return carry

    lax.fori_loop(0, 16, zrow, 0)
    for k in range(SPT // 16):
        pltpu.sync_copy(obuf, s_sh.at[pl.ds(sid * SPT + k * 16, 16)])
    pltpu.sync_copy(obuf.at[pl.ds(0, SPT % 16)],
                    s_sh.at[pl.ds(sid * SPT + (SPT // 16) * 16, SPT % 16)])
    for k in range(DPT // 16):
        pltpu.sync_copy(obuf,
                        s_sh.at[pl.ds(S_ROWS + sid * DPT + k * 16, 16)])
    plsc.subcore_barrier()

    pltpu.sync_copy(gmax_hbm.at[pl.ds(0, 16)], gbuf.at[0])

    def bsl(s):
        return pl.ds(s * CH, CH)

    def slot(c):
        return lax.rem(c, 4)

    def fetch(c, s):
        base = wid * EPT + c * CH
        pltpu.async_copy(att_hbm.at[pl.ds(base, CH)], attb.at[s], isem[s])
        pltpu.async_copy(head_hbm.at[pl.ds(base, CH)],
                         hidx.at[s].at[pl.ds(0, CH)], isem[s])
        pltpu.async_copy(head_hbm.at[pl.ds(base, CH)], hc_all.at[slot(c)],
                         isem[s])
        pltpu.async_copy(v_hbm.at[pl.ds(base, CH)], vbuf.at[bsl(s)], isem[s])

    def wait_in(s):
        pltpu.make_async_copy(att_hbm.at[pl.ds(0, CH)], attb.at[s],
                              isem[s]).wait()
        pltpu.make_async_copy(head_hbm.at[pl.ds(0, CH)],
                              hidx.at[s].at[pl.ds(0, CH)], isem[s]).wait()
        pltpu.make_async_copy(head_hbm.at[pl.ds(0, CH)], hc_all.at[0],
                              isem[s]).wait()
        pltpu.make_async_copy(v_hbm.at[pl.ds(0, CH)], vbuf.at[bsl(s)],
                              isem[s]).wait()

    def wait_adds(s):
        pltpu.make_async_copy(v_hbm.at[pl.ds(0, CH)], sbuf.at[bsl(s)],
                              asem[s]).wait()
        pltpu.make_async_copy(v_hbm.at[pl.ds(0, CH)], rows2.at[bsl(s)],
                              asem[s]).wait()

    def compute(c, s):
        base = wid * EPT + c * CH
        dr = slot(c)
        g = gbuf[0]
        for gi in range(CH // 16):
            sl = pl.ds(16 * gi, 16)
            a = attb[s, sl]
            eid = base + gi * 16 + lanes
            valid = jnp.where(eid < N_EDGES, 1.0, 0.0).astype(F32)
            wbuf[s, sl] = jnp.exp(a - g) * valid
            valbuf[s, sl] = valid
            h16 = hidx[s, sl]
            hh_all[dr, sl] = S_ROWS + lax.shift_right_logical(h16, 2)

        @functools.partial(plsc.parallel_loop, 0, CH, unroll=4)
        def row(r):
            wj = wbuf[s, pl.ds(r, 16)][0]
            vj = valbuf[s, pl.ds(r, 16)][0]
            hj = hidx[s, pl.ds(r, 16)][0]
            b = lax.rem(hj, 4)
            wv = jnp.full((16,), wj, F32)
            rr = s * CH + r
            for q in range(EMB // 16):
                sl = pl.ds(16 * q, 16)
                sbuf[rr, sl] = vbuf[rr, sl] * wv
                ff = (b == (q // 2)).astype(F32)
                sj = (wj if q % 2 == 0 else vj) * ff
                rows2[rr, sl] = jnp.full((16,), sj, F32)

    def start_adds(c, s):
        dr = slot(c)
        pltpu.async_copy(sbuf.at[bsl(s)], s_sh.at[hc_all.at[dr]], asem[s],
                         add=True)
        pltpu.async_copy(rows2.at[bsl(s)], s_sh.at[hh_all.at[dr]], asem[s],
                         add=True)

    fetch(0, 0)
    fetch(1, 1)
    for c in (0, 1):
        wait_in(c)
        compute(c, c)
        start_adds(c, c)
        fetch(c + 2, c)

    def main(c2, carry):
        for s in (0, 1):
            c = 2 * c2 + s
            wait_in(s)
            wait_adds(s)
            compute(c, s)
            start_adds(c, s)
            fetch(c + 2, s)
        return carry

    lax.fori_loop(1, CHUNKS // 2 - 1, main, 0)
    for s in (0, 1):
        c = CHUNKS - 2 + s
        wait_in(s)
        wait_adds(s)
        compute(c, s)
        start_adds(c, s)
    wait_adds(0)
    wait_adds(1)
    plsc.subcore_barrier()

    def unp_round(rnd, carry):
        pltpu.sync_copy(s_sh.at[pl.ds(S_ROWS + sid * DPT + rnd * 8, 8)],
                        vbuf.at[pl.ds(0, 8)])
        for half in range(2):
            for dr in range(4):
                for slot in range(4):
                    src_r = half * 4 + dr
                    den_s = vbuf[src_r, pl.ds(32 * slot, 16)][0]
                    cnt_s = vbuf[src_r, pl.ds(32 * slot + 16, 16)][0]
                    tv = jnp.where(
                        lanes == 0, jnp.full((16,), den_s, F32),
                        jnp.where(lanes == 1, jnp.full((16,), cnt_s, F32),
                                  zero16))
                    obuf[dr * 4 + slot, pl.ds(0, 16)] = tv
            pltpu.sync_copy(
                obuf,
                d2_out.at[cid].at[pl.ds(sid * EPTILE + rnd * 32 + half * 16,
                                        16)])
        return carry

    lax.fori_loop(0, DPT // 8, unp_round, 0)

    sl = pl.ds(sid * SPT, SPT)
    pltpu.sync_copy(s_sh.at[sl], s2_out.at[cid].at[sl])


_sc_scatter = functools.partial(
    pl.kernel,
    out_type=(jax.ShapeDtypeStruct((NC, N_ENT_PAD, EMB), F32),
              jax.ShapeDtypeStruct((NC, N_ENT_PAD, EMB), F32)),
    mesh=plsc.VectorSubcoreMesh(core_axis_name="c", subcore_axis_name="s"),
    scratch_types=[
        pltpu.VMEM((2, CH), F32),
        pltpu.VMEM((2, CH + 16), jnp.int32),
        pltpu.VMEM((4, CH), jnp.int32),
        pltpu.VMEM((4, CH), jnp.int32),
        pltpu.VMEM((1, 16), F32),
        pltpu.VMEM((2, CH + 16), F32),
        pltpu.VMEM((2, CH + 16), F32),
        pltpu.VMEM((2 * CH, EMB), F32),
        pltpu.VMEM((2 * CH, EMB), F32),
        pltpu.VMEM((2 * CH, EMB), F32),
        pltpu.VMEM((16, EMB), F32),
        pltpu.VMEM_SHARED((S_ROWS + D_ROWS, EMB), F32),
        pltpu.SemaphoreType.DMA,
        pltpu.SemaphoreType.DMA,
        pltpu.SemaphoreType.DMA,
        pltpu.SemaphoreType.DMA,
    ],
)(_sc_scatter_body)


def _tc_att_body(v_ref, qh_ref, kw_ref, att_ref, gm_ref):
    right = jnp.tanh(jnp.dot(v_ref[...], kw_ref[...],
                             preferred_element_type=F32))
    s = jnp.sum(qh_ref[...] * right, axis=1)
    att_ref[0] = s.reshape(8, 128)
    m = jnp.max(s)

    @pl.when(pl.program_id(0) == 0)
    def _():
        gm_ref[...] = jnp.full((8, 128), -3e38, F32)

    gm_ref[...] = jnp.maximum(gm_ref[...], m)


def _tc_att(v, qh, k_w):
    return pl.pallas_call(
        _tc_att_body,
        grid=(NB,),
        in_specs=[
            pl.BlockSpec((BLK, EMB), lambda i: (i, 0)),
            pl.BlockSpec((BLK, EMB), lambda i: (i, 0)),
            pl.BlockSpec((EMB, EMB), lambda i: (0, 0)),
        ],
        out_specs=[
            pl.BlockSpec((1, 8, 128), lambda i: (i, 0, 0)),
            pl.BlockSpec((8, 128), lambda i: (0, 0)),
        ],
        out_shape=[
            jax.ShapeDtypeStruct((NB, 8, 128), F32),
            jax.ShapeDtypeStruct((8, 128), F32),
        ],
    )(v, qh, k_w)


def _tc_combine_body(s2_ref, d2_ref, ee_ref, prev_ref, qw_ref,
                     out_ref, agg_ref, q_ref):
    sv = s2_ref[0] + s2_ref[1]
    dd = d2_ref[0] + d2_ref[1]
    den = dd[:, 0:1] + 1e-16
    cnt = jnp.maximum(dd[:, 1:2], 1.0)
    aggv = sv / den / cnt
    n2 = jnp.sum(aggv * aggv, axis=1, keepdims=True)
    aggn = aggv / jnp.maximum(jnp.sqrt(n2), 1e-12)
    out_ref[...] = prev_ref[...] + aggn + ee_ref[...]
    agg_ref[...] = aggn
    q_ref[...] = jnp.dot(aggn, qw_ref[...], preferred_element_type=F32)


def _tc_combine(s2, d2, ee_pad, prev, q_w):
    return pl.pallas_call(
        _tc_combine_body,
        grid=(N_ENT_PAD // RB,),
        in_specs=[
            pl.BlockSpec((NC, RB, EMB), lambda i: (0, i, 0)),
            pl.BlockSpec((NC, RB, EMB), lambda i: (0, i, 0)),
            pl.BlockSpec((RB, EMB), lambda i: (i, 0)),
            pl.BlockSpec((RB, EMB), lambda i: (i, 0)),
            pl.BlockSpec((EMB, EMB), lambda i: (0, 0)),
        ],
        out_specs=[
            pl.BlockSpec((RB, EMB), lambda i: (i, 0)),
            pl.BlockSpec((RB, EMB), lambda i: (i, 0)),
            pl.BlockSpec((RB, EMB), lambda i: (i, 0)),
        ],
        out_shape=[
            jax.ShapeDtypeStruct((N_ENT_PAD, EMB), F32),
            jax.ShapeDtypeStruct((N_ENT_PAD, EMB), F32),
            jax.ShapeDtypeStruct((N_ENT_PAD, EMB), F32),
        ],
    )(s2, d2, ee_pad, prev, q_w)


def _tc_q_body(x_ref, qw_ref, q_ref):
    q_ref[...] = jnp.dot(x_ref[...], qw_ref[...], preferred_element_type=F32)


def _tc_q(x, q_w):
    return pl.pallas_call(
        _tc_q_body,
        grid=(N_ENT_PAD // RB,),
        in_specs=[
            pl.BlockSpec((RB, EMB), lambda i: (i, 0)),
            pl.BlockSpec((EMB, EMB), lambda i: (0, 0)),
        ],
        out_specs=pl.BlockSpec((RB, EMB), lambda i: (i, 0)),
        out_shape=jax.ShapeDtypeStruct((N_ENT_PAD, EMB), F32),
    )(x, q_w)


def kernel(entity_emb, edge_index, edge_type, edge_emb, mess_dropout,
           q_w, k_w):
    ee = entity_emb.astype(F32)
    head = edge_index[0].astype(jnp.int32)
    tail = edge_index[1].astype(jnp.int32)
    typ = edge_type.astype(jnp.int32)
    pad = E_PAD - N_EDGES
    head_p = jnp.concatenate([head, jnp.zeros((pad,), jnp.int32)])
    tail_p = jnp.concatenate([tail, jnp.zeros((pad,), jnp.int32)])
    typ_p = jnp.concatenate([typ, jnp.zeros((pad,), jnp.int32)])

    ee_pad = jnp.concatenate(
        [ee, jnp.zeros((N_ENT_PAD - N_ENT, EMB), F32)], axis=0)
    out = jnp.zeros((N_ENT_PAD, EMB), F32)
    agg = ee_pad
    q = _tc_q(ee_pad, q_w)
    for _ in range(N_HOPS):
        v, qh = _sc_gather(agg, q, tail_p, head_p, typ_p, edge_emb)
        att3, gm = _tc_att(v, qh, k_w)
        s2, d2 = _sc_scatter(att3.reshape(E_PAD), gm.reshape(BLK), head_p, v)
        out, agg, q = _tc_combine(s2, d2, ee_pad, out, q_w)
    return out[:N_ENT]
